# Initial kernel scaffold; baseline (speedup 1.0000x reference)
import jax
import jax.numpy as jnp
from jax.experimental import pallas as pl


def kernel(pos, x, z, edge_index, edge_vec, Wsc0, Wlin10, Wfc10, Wfc20, Wlin20, Wsc1, Wlin11, Wfc11, Wfc21, Wlin21, Wsc2, Wlin12, Wfc12, Wfc22, Wlin22):
    raise NotImplementedError("write your pallas kernel here")


# speedup vs baseline: 2.9715x; 2.9715x over previous
import functools
import math

import jax
import jax.numpy as jnp
from jax import lax
from jax.experimental import pallas as pl
from jax.experimental.pallas import tpu as pltpu
from jax.experimental.pallas import tpu_sc as plsc

N = 50000
E = 1600000
D = 16
NB = 10
RN = 100
MAX_RADIUS = 3.5
SIN_C = math.sin(math.pi / 8)
COS_C = math.cos(math.pi / 8)
INV_SQRT_NEIGH = 1.0 / math.sqrt(32.0)
INV_SQRT_D = 1.0 / math.sqrt(float(D))
INV_SQRT_NB = 1.0 / math.sqrt(float(NB))
INV_SQRT_RN = 1.0 / math.sqrt(float(RN))

NUM_CORES = 2
NUM_SUBCORES = 16
NW = NUM_CORES * NUM_SUBCORES
EPW = E // NW
SUB = 125
CHUNK = 1000
SUBS_PER_CHUNK = CHUNK // SUB
CHUNKS = EPW // CHUNK
ROWS_PER_TILE = 3128
NP = NUM_SUBCORES * ROWS_PER_TILE

BE = 8000
EBLOCKS = E // BE
BN = NP // 8


def _edge_mlp_body(ev_ref, w10, w20, w11, w21, w12, w22, o0, o1, o2):
    ev = ev_ref[...]
    lv = jnp.sqrt(jnp.sum(ev * ev, axis=1))
    step = MAX_RADIUS / (NB - 1)
    vals = lax.broadcasted_iota(jnp.int32, (1, NB), 1).astype(jnp.float32) * step
    diff = (lv[:, None] - vals) * (1.0 / step)
    emb = jnp.exp(-(diff * diff)) * (math.sqrt(float(NB)) / 1.12)
    u = 2.0 * (lv * (1.0 / MAX_RADIUS) - 1.0)
    y = (1.0 - jnp.cos(math.pi * u)) * 0.5
    y = jnp.where(u > 0.0, 0.0, y)
    y = jnp.where(u < -1.0, 1.0, y)
    attr = y[:, None]
    for w1, w2, o in ((w10, w20, o0), (w11, w21, o1), (w12, w22, o2)):
        t = jnp.dot(emb, w1[...], preferred_element_type=jnp.float32) * INV_SQRT_NB
        t = t * jax.nn.sigmoid(t)
        w = jnp.dot(t, w2[...], preferred_element_type=jnp.float32) * INV_SQRT_RN
        o[...] = w * attr


def _edge_mlp(edge_vec, Wfc10, Wfc20, Wfc11, Wfc21, Wfc12, Wfc22):
    wshape = jax.ShapeDtypeStruct((E, D), jnp.float32)
    full = lambda s: pl.BlockSpec(s, lambda i: (0, 0))
    return pl.pallas_call(
        _edge_mlp_body,
        grid=(EBLOCKS,),
        in_specs=[
            pl.BlockSpec((BE, 3), lambda i: (i, 0)),
            full((NB, RN)), full((RN, D)),
            full((NB, RN)), full((RN, D)),
            full((NB, RN)), full((RN, D)),
        ],
        out_specs=[pl.BlockSpec((BE, D), lambda i: (i, 0))] * 3,
        out_shape=[wshape, wshape, wshape],
    )(edge_vec, Wfc10, Wfc20, Wfc11, Wfc21, Wfc12, Wfc22)


def _node_pre_body(x_ref, z_ref, wsc_ref, wlin1_ref, s_ref, h_ref):
    xz = x_ref[...] * z_ref[...]
    s_ref[...] = jnp.dot(xz, wsc_ref[...], preferred_element_type=jnp.float32) * INV_SQRT_D
    h_ref[...] = jnp.dot(xz, wlin1_ref[...], preferred_element_type=jnp.float32) * INV_SQRT_D


def _node_pre(x, z, Wsc, Wlin1):
    o = jax.ShapeDtypeStruct((NP, D), jnp.float32)
    full = lambda s: pl.BlockSpec(s, lambda i: (0, 0))
    return pl.pallas_call(
        _node_pre_body,
        grid=(NP // BN,),
        in_specs=[
            pl.BlockSpec((BN, D), lambda i: (i, 0)),
            pl.BlockSpec((BN, 1), lambda i: (i, 0)),
            full((D, D)), full((D, D)),
        ],
        out_specs=[pl.BlockSpec((BN, D), lambda i: (i, 0))] * 2,
        out_shape=[o, o],
    )(x, z, Wsc, Wlin1)


def _node_post_body(agg2_ref, z_ref, wlin2_ref, s_ref, x_ref, *, do_silu):
    a = (agg2_ref[0] + agg2_ref[1]) * INV_SQRT_NEIGH
    o = jnp.dot(a * z_ref[...], wlin2_ref[...],
                preferred_element_type=jnp.float32) * INV_SQRT_D
    xn = SIN_C * s_ref[...] + COS_C * o
    if do_silu:
        xn = xn * jax.nn.sigmoid(xn)
    x_ref[...] = xn


def _node_post(agg2, z, Wlin2, s, do_silu):
    return pl.pallas_call(
        functools.partial(_node_post_body, do_silu=do_silu),
        grid=(NP // BN,),
        in_specs=[
            pl.BlockSpec((2, BN, D), lambda i: (0, i, 0)),
            pl.BlockSpec((BN, 1), lambda i: (i, 0)),
            pl.BlockSpec((D, D), lambda i: (0, 0)),
            pl.BlockSpec((BN, D), lambda i: (i, 0)),
        ],
        out_specs=pl.BlockSpec((BN, D), lambda i: (i, 0)),
        out_shape=jax.ShapeDtypeStruct((NP, D), jnp.float32),
    )(agg2, z, Wlin2, s)


def _sc_body(h_hbm, wa_hbm, src_hbm, dst_hbm, out_hbm,
             src_v, dst_v, wa_v, rows_v, agg_sh, sem):
    c = lax.axis_index("c")
    s = lax.axis_index("s")
    w = c * NUM_SUBCORES + s

    def _zero(i, _):
        rows_v[i] = jnp.zeros((D,), jnp.float32)
        return 0
    lax.fori_loop(0, CHUNK, _zero, 0)
    zrow = pl.multiple_of(s * ROWS_PER_TILE, 8)
    for r in range(ROWS_PER_TILE // CHUNK):
        pltpu.sync_copy(rows_v, agg_sh.at[pl.ds(zrow + r * CHUNK, CHUNK)])
    rem = ROWS_PER_TILE % CHUNK
    if rem:
        pltpu.sync_copy(rows_v.at[pl.ds(0, rem)],
                        agg_sh.at[pl.ds(zrow + (ROWS_PER_TILE // CHUNK) * CHUNK, rem)])
    plsc.subcore_barrier()

    def _chunk(ci, _):
        edge_base = pl.multiple_of(w * EPW + ci * CHUNK, 8)
        row_base = pl.multiple_of(edge_base // SUB, 8)
        pltpu.sync_copy(src_hbm.at[pl.ds(row_base, SUBS_PER_CHUNK)], src_v)
        pltpu.sync_copy(dst_hbm.at[pl.ds(row_base, SUBS_PER_CHUNK)], dst_v)
        pltpu.sync_copy(wa_hbm.at[pl.ds(edge_base, CHUNK)], wa_v)
        for j in range(SUBS_PER_CHUNK):
            pltpu.async_copy(h_hbm.at[src_v.at[j]],
                             rows_v.at[pl.ds(j * SUB, SUB)], sem).wait()

        def _mul(i, _):
            rows_v[i] = rows_v[i] * wa_v[i]
            return 0
        lax.fori_loop(0, CHUNK, _mul, 0)

        for j in range(SUBS_PER_CHUNK):
            pltpu.sync_copy(rows_v.at[pl.ds(j * SUB, SUB)],
                            agg_sh.at[dst_v.at[j]], add=True)
        return 0

    lax.fori_loop(0, CHUNKS, _chunk, 0)
    plsc.subcore_barrier()
    orow = pl.multiple_of(s * ROWS_PER_TILE, 8)
    pltpu.sync_copy(agg_sh.at[pl.ds(orow, ROWS_PER_TILE)],
                    out_hbm.at[c].at[pl.ds(orow, ROWS_PER_TILE)])


@functools.lru_cache(maxsize=None)
def _get_sc_kernel():
    return functools.partial(
        pl.kernel,
        mesh=plsc.VectorSubcoreMesh(
            core_axis_name="c", subcore_axis_name="s",
            num_cores=NUM_CORES, num_subcores=NUM_SUBCORES),
        out_type=jax.ShapeDtypeStruct((NUM_CORES, NP, D), jnp.float32),
        scratch_types=[
            pltpu.VMEM((SUBS_PER_CHUNK, SUB), jnp.int32),
            pltpu.VMEM((SUBS_PER_CHUNK, SUB), jnp.int32),
            pltpu.VMEM((CHUNK, D), jnp.float32),
            pltpu.VMEM((CHUNK, D), jnp.float32),
            pltpu.VMEM_SHARED((NP, D), jnp.float32),
            pltpu.SemaphoreType.DMA,
        ],
        compiler_params=pltpu.CompilerParams(use_tc_tiling_on_sc=False),
    )(_sc_body)


def _sc_gather_scatter(h, wa, src2, dst2):
    return _get_sc_kernel()(h, wa, src2, dst2)


def kernel(pos, x, z, edge_index, edge_vec,
           Wsc0, Wlin10, Wfc10, Wfc20, Wlin20,
           Wsc1, Wlin11, Wfc11, Wfc21, Wlin21,
           Wsc2, Wlin12, Wfc12, Wfc22, Wlin22):
    del pos
    src2 = edge_index[0].reshape(E // SUB, SUB)
    dst2 = edge_index[1].reshape(E // SUB, SUB)
    x = jnp.pad(x, ((0, NP - N), (0, 0)))
    z = jnp.pad(z, ((0, NP - N), (0, 0)))
    wa0, wa1, wa2 = _edge_mlp(edge_vec, Wfc10, Wfc20, Wfc11, Wfc21, Wfc12, Wfc22)

    layers = (
        (Wsc0, Wlin10, Wlin20, wa0, True),
        (Wsc1, Wlin11, Wlin21, wa1, True),
        (Wsc2, Wlin12, Wlin22, wa2, False),
    )
    for Wsc, Wlin1, Wlin2, wa, do_silu in layers:
        s, h = _node_pre(x, z, Wsc, Wlin1)
        agg2 = _sc_gather_scatter(h, wa, src2, dst2)
        x = _node_post(agg2, z, Wlin2, s, do_silu)
    return x[:N]


# speedup vs baseline: 3.4976x; 1.1771x over previous
import functools
import math

import jax
import jax.numpy as jnp
from jax import lax
from jax.experimental import pallas as pl
from jax.experimental.pallas import tpu as pltpu
from jax.experimental.pallas import tpu_sc as plsc

N = 50000
E = 1600000
D = 16
NB = 10
RN = 100
MAX_RADIUS = 3.5
SIN_C = math.sin(math.pi / 8)
COS_C = math.cos(math.pi / 8)
INV_SQRT_NEIGH = 1.0 / math.sqrt(32.0)
INV_SQRT_D = 1.0 / math.sqrt(float(D))
INV_SQRT_NB = 1.0 / math.sqrt(float(NB))
INV_SQRT_RN = 1.0 / math.sqrt(float(RN))

NUM_CORES = 2
NUM_SUBCORES = 16
NW = NUM_CORES * NUM_SUBCORES
EPW = E // NW
SUB = 125
CHUNK = 1000
SUBS_PER_CHUNK = CHUNK // SUB
CHUNKS = EPW // CHUNK
ROWS_PER_TILE = 3128
NP = NUM_SUBCORES * ROWS_PER_TILE

BE = 8000
EBLOCKS = E // BE
BN = NP // 8


def _edge_geom_body(vx_ref, vy_ref, vz_ref, lv_ref, y_ref):
    vx, vy, vz = vx_ref[...], vy_ref[...], vz_ref[...]
    lv = jnp.sqrt(vx * vx + vy * vy + vz * vz)
    u = 2.0 * (lv * (1.0 / MAX_RADIUS) - 1.0)
    y = (1.0 - jnp.cos(math.pi * u)) * 0.5
    y = jnp.where(u > 0.0, 0.0, y)
    y = jnp.where(u < -1.0, 1.0, y)
    lv_ref[...] = lv
    y_ref[...] = y


GROWS = E // 128


def _edge_geom(vx, vy, vz):
    o = jax.ShapeDtypeStruct((GROWS, 128), jnp.float32)
    return pl.pallas_call(_edge_geom_body, out_shape=[o, o])(vx, vy, vz)


def _edge_mlp_body(lv_ref, y_ref, w10, w20, w11, w21, w12, w22, o0, o1, o2):
    lv = lv_ref[...]
    step = MAX_RADIUS / (NB - 1)
    vals = lax.broadcasted_iota(jnp.int32, (1, NB), 1).astype(jnp.float32) * step
    diff = (lv - vals) * (1.0 / step)
    emb = jnp.exp(-(diff * diff)) * (math.sqrt(float(NB)) / 1.12)
    attr = y_ref[...]
    for w1, w2, o in ((w10, w20, o0), (w11, w21, o1), (w12, w22, o2)):
        t = jnp.dot(emb, w1[...], preferred_element_type=jnp.float32) * INV_SQRT_NB
        t = (0.5 * t) * (1.0 + jnp.tanh(0.5 * t))
        w = jnp.dot(t, w2[...], preferred_element_type=jnp.float32) * INV_SQRT_RN
        o[...] = w * attr


def _edge_mlp(lv1, y1, Wfc10, Wfc20, Wfc11, Wfc21, Wfc12, Wfc22):
    wshape = jax.ShapeDtypeStruct((E, D), jnp.float32)
    full = lambda s: pl.BlockSpec(s, lambda i: (0, 0))
    return pl.pallas_call(
        _edge_mlp_body,
        grid=(EBLOCKS,),
        in_specs=[
            pl.BlockSpec((BE, 1), lambda i: (i, 0)),
            pl.BlockSpec((BE, 1), lambda i: (i, 0)),
            full((NB, RN)), full((RN, D)),
            full((NB, RN)), full((RN, D)),
            full((NB, RN)), full((RN, D)),
        ],
        out_specs=[pl.BlockSpec((BE, D), lambda i: (i, 0))] * 3,
        out_shape=[wshape, wshape, wshape],
    )(lv1, y1, Wfc10, Wfc20, Wfc11, Wfc21, Wfc12, Wfc22)


def _node_pre_body(x_ref, z_ref, wsc_ref, wlin1_ref, s_ref, h_ref):
    xz = x_ref[...] * z_ref[...]
    s_ref[...] = jnp.dot(xz, wsc_ref[...], preferred_element_type=jnp.float32) * INV_SQRT_D
    h_ref[...] = jnp.dot(xz, wlin1_ref[...], preferred_element_type=jnp.float32) * INV_SQRT_D


def _node_pre(x, z, Wsc, Wlin1):
    o = jax.ShapeDtypeStruct((NP, D), jnp.float32)
    full = lambda s: pl.BlockSpec(s, lambda i: (0, 0))
    return pl.pallas_call(
        _node_pre_body,
        grid=(NP // BN,),
        in_specs=[
            pl.BlockSpec((BN, D), lambda i: (i, 0)),
            pl.BlockSpec((BN, 1), lambda i: (i, 0)),
            full((D, D)), full((D, D)),
        ],
        out_specs=[pl.BlockSpec((BN, D), lambda i: (i, 0))] * 2,
        out_shape=[o, o],
    )(x, z, Wsc, Wlin1)


def _node_post_body(agg2_ref, z_ref, wlin2_ref, s_ref, x_ref, *, do_silu):
    a = (agg2_ref[0] + agg2_ref[1]) * INV_SQRT_NEIGH
    o = jnp.dot(a * z_ref[...], wlin2_ref[...],
                preferred_element_type=jnp.float32) * INV_SQRT_D
    xn = SIN_C * s_ref[...] + COS_C * o
    if do_silu:
        xn = (0.5 * xn) * (1.0 + jnp.tanh(0.5 * xn))
    x_ref[...] = xn


def _node_post(agg2, z, Wlin2, s, do_silu):
    return pl.pallas_call(
        functools.partial(_node_post_body, do_silu=do_silu),
        grid=(NP // BN,),
        in_specs=[
            pl.BlockSpec((2, BN, D), lambda i: (0, i, 0)),
            pl.BlockSpec((BN, 1), lambda i: (i, 0)),
            pl.BlockSpec((D, D), lambda i: (0, 0)),
            pl.BlockSpec((BN, D), lambda i: (i, 0)),
        ],
        out_specs=pl.BlockSpec((BN, D), lambda i: (i, 0)),
        out_shape=jax.ShapeDtypeStruct((NP, D), jnp.float32),
    )(agg2, z, Wlin2, s)


def _sc_body(h_hbm, wa_hbm, src_hbm, dst_hbm, out_hbm,
             src_v, dst_v, wa_v, rows_v, agg_sh, fsem0, fsem1, gsem0, gsem1):
    c = lax.axis_index("c")
    s = lax.axis_index("s")
    w = c * NUM_SUBCORES + s
    fsems = (fsem0, fsem1)
    gsems = (gsem0, gsem1)

    def _zero(i, _):
        rows_v[0, i] = jnp.zeros((D,), jnp.float32)
        return 0
    lax.fori_loop(0, CHUNK, _zero, 0)
    zrow = pl.multiple_of(s * ROWS_PER_TILE, 8)
    for r in range(ROWS_PER_TILE // CHUNK):
        pltpu.sync_copy(rows_v.at[0], agg_sh.at[pl.ds(zrow + r * CHUNK, CHUNK)])
    rem = ROWS_PER_TILE % CHUNK
    if rem:
        pltpu.sync_copy(rows_v.at[0].at[pl.ds(0, rem)],
                        agg_sh.at[pl.ds(zrow + (ROWS_PER_TILE // CHUNK) * CHUNK, rem)])
    plsc.subcore_barrier()

    def _start_fetch(chunk, b):
        edge_base = pl.multiple_of(w * EPW + chunk * CHUNK, 8)
        row_base = pl.multiple_of(edge_base // SUB, 8)
        pltpu.async_copy(src_hbm.at[pl.ds(row_base, SUBS_PER_CHUNK)],
                         src_v.at[b], fsems[b])
        pltpu.async_copy(dst_hbm.at[pl.ds(row_base, SUBS_PER_CHUNK)],
                         dst_v.at[b], fsems[b])
        pltpu.async_copy(wa_hbm.at[pl.ds(edge_base, CHUNK)], wa_v.at[b], fsems[b])

    def _drain_fetch(b):
        pltpu.make_async_copy(src_hbm.at[pl.ds(0, SUBS_PER_CHUNK)],
                              src_v.at[b], fsems[b]).wait()
        pltpu.make_async_copy(dst_hbm.at[pl.ds(0, SUBS_PER_CHUNK)],
                              dst_v.at[b], fsems[b]).wait()
        pltpu.make_async_copy(wa_hbm.at[pl.ds(0, CHUNK)], wa_v.at[b],
                              fsems[b]).wait()

    def _fire_gathers(b):
        for j in range(SUBS_PER_CHUNK):
            pltpu.async_copy(h_hbm.at[src_v.at[b].at[j]],
                             rows_v.at[b].at[pl.ds(j * SUB, SUB)], gsems[b])

    def _wait_gathers(b):
        for j in range(SUBS_PER_CHUNK):
            pltpu.make_async_copy(h_hbm.at[src_v.at[b].at[j]],
                                  rows_v.at[b].at[pl.ds(j * SUB, SUB)],
                                  gsems[b]).wait()

    _start_fetch(0, 0)
    _drain_fetch(0)
    _fire_gathers(0)
    _start_fetch(1, 1)

    def _pair(ci, _):
        for b in range(2):
            chunk = 2 * ci + b
            _wait_gathers(b)

            @pl.when(chunk + 1 < CHUNKS)
            def _():
                _drain_fetch(1 - b)
                _fire_gathers(1 - b)

            def _mul(i, _):
                for k in range(4):
                    r = 4 * i + k
                    rows_v[b, r] = rows_v[b, r] * wa_v[b, r]
                return 0
            lax.fori_loop(0, CHUNK // 4, _mul, 0)

            for j in range(SUBS_PER_CHUNK):
                pltpu.sync_copy(rows_v.at[b].at[pl.ds(j * SUB, SUB)],
                                agg_sh.at[dst_v.at[b].at[j]], add=True)

            @pl.when(chunk + 2 < CHUNKS)
            def _():
                _start_fetch(chunk + 2, b)
        return 0

    lax.fori_loop(0, CHUNKS // 2, _pair, 0)
    plsc.subcore_barrier()
    orow = pl.multiple_of(s * ROWS_PER_TILE, 8)
    pltpu.sync_copy(agg_sh.at[pl.ds(orow, ROWS_PER_TILE)],
                    out_hbm.at[c].at[pl.ds(orow, ROWS_PER_TILE)])


@functools.lru_cache(maxsize=None)
def _get_sc_kernel():
    return functools.partial(
        pl.kernel,
        mesh=plsc.VectorSubcoreMesh(
            core_axis_name="c", subcore_axis_name="s",
            num_cores=NUM_CORES, num_subcores=NUM_SUBCORES),
        out_type=jax.ShapeDtypeStruct((NUM_CORES, NP, D), jnp.float32),
        scratch_types=[
            pltpu.VMEM((2, SUBS_PER_CHUNK, SUB), jnp.int32),
            pltpu.VMEM((2, SUBS_PER_CHUNK, SUB), jnp.int32),
            pltpu.VMEM((2, CHUNK, D), jnp.float32),
            pltpu.VMEM((2, CHUNK, D), jnp.float32),
            pltpu.VMEM_SHARED((NP, D), jnp.float32),
            pltpu.SemaphoreType.DMA,
            pltpu.SemaphoreType.DMA,
            pltpu.SemaphoreType.DMA,
            pltpu.SemaphoreType.DMA,
        ],
        compiler_params=pltpu.CompilerParams(use_tc_tiling_on_sc=False),
    )(_sc_body)


def _sc_gather_scatter(h, wa, src2, dst2):
    return _get_sc_kernel()(h, wa, src2, dst2)


def kernel(pos, x, z, edge_index, edge_vec,
           Wsc0, Wlin10, Wfc10, Wfc20, Wlin20,
           Wsc1, Wlin11, Wfc11, Wfc21, Wlin21,
           Wsc2, Wlin12, Wfc12, Wfc22, Wlin22):
    del pos
    src2 = edge_index[0].reshape(E // SUB, SUB)
    dst2 = edge_index[1].reshape(E // SUB, SUB)
    x = jnp.pad(x, ((0, NP - N), (0, 0)))
    z = jnp.pad(z, ((0, NP - N), (0, 0)))
    vx = edge_vec[:, 0].reshape(GROWS, 128)
    vy = edge_vec[:, 1].reshape(GROWS, 128)
    vz = edge_vec[:, 2].reshape(GROWS, 128)
    lvp, yp = _edge_geom(vx, vy, vz)
    wa0, wa1, wa2 = _edge_mlp(lvp.reshape(E, 1), yp.reshape(E, 1),
                              Wfc10, Wfc20, Wfc11, Wfc21, Wfc12, Wfc22)

    layers = (
        (Wsc0, Wlin10, Wlin20, wa0, True),
        (Wsc1, Wlin11, Wlin21, wa1, True),
        (Wsc2, Wlin12, Wlin22, wa2, False),
    )
    for Wsc, Wlin1, Wlin2, wa, do_silu in layers:
        s, h = _node_pre(x, z, Wsc, Wlin1)
        agg2 = _sc_gather_scatter(h, wa, src2, dst2)
        x = _node_post(agg2, z, Wlin2, s, do_silu)
    return x[:N]
